# baseline (speedup 1.0000x reference)
import functools

import jax
import jax.numpy as jnp
from jax import lax
from jax.experimental import pallas as pl
from jax.experimental.pallas import tpu as pltpu
from jax.experimental.pallas import tpu_sc as plsc

NUM_OUT = 1024
NUM_WORKERS = 32
ROWS_PER_WORKER = NUM_OUT // NUM_WORKERS


def _sc_subsample(landmarks):
    t, r, c = landmarks.shape
    num = 2 * (t - 1)
    den = 2 * (NUM_OUT - 1)
    half = NUM_OUT - 1
    mesh = plsc.VectorSubcoreMesh(core_axis_name="c", subcore_axis_name="s")

    @functools.partial(
        pl.kernel,
        mesh=mesh,
        out_type=jax.ShapeDtypeStruct((NUM_OUT, r, c), landmarks.dtype),
        scratch_types=[pltpu.SemaphoreType.DMA],
    )
    def k(table_hbm, out_hbm, sem):
        wid = lax.axis_index("s") * 2 + lax.axis_index("c")
        base = wid * ROWS_PER_WORKER
        copies = []
        for j in range(ROWS_PER_WORKER):
            i = base + j
            idx = (num * i + half) // den
            copies.append(
                pltpu.async_copy(
                    table_hbm.at[pl.ds(idx, 1)], out_hbm.at[pl.ds(i, 1)], sem
                )
            )
        for cp in copies:
            cp.wait()

    return k(landmarks)


def kernel(landmarks):
    return _sc_subsample(landmarks)


# speedup vs baseline: 66.5972x; 66.5972x over previous
import functools

import jax
import jax.numpy as jnp
from jax import lax
from jax.experimental import pallas as pl
from jax.experimental.pallas import tpu as pltpu
from jax.experimental.pallas import tpu_sc as plsc

NUM_OUT = 1024
NUM_WORKERS = 32


def _sc_lane_subsample(x_t):
    c, r, t = x_t.shape
    m_rows = c * r
    rpw = -(-m_rows // NUM_WORKERS)
    nrows = rpw + (rpw % 2)
    last = m_rows - 1
    num = 2 * (t - 1)
    den = 2 * (NUM_OUT - 1)
    half = NUM_OUT - 1
    mesh = plsc.VectorSubcoreMesh(core_axis_name="c", subcore_axis_name="s")

    @functools.partial(
        pl.kernel,
        mesh=mesh,
        out_type=jax.ShapeDtypeStruct((c, r, NUM_OUT), x_t.dtype),
        scratch_types=[
            pltpu.VMEM((2, 1, t), x_t.dtype),
            pltpu.VMEM((2, 1, NUM_OUT), x_t.dtype),
            pltpu.SemaphoreType.DMA,
            pltpu.SemaphoreType.DMA,
        ],
        compiler_params=pltpu.CompilerParams(needs_layout_passes=False),
    )
    def k(x_hbm, out_hbm, rowbuf, outbuf, isem, osem):
        wid = lax.axis_index("s") * 2 + lax.axis_index("c")
        base = wid * rpw
        lane = lax.iota(jnp.int32, 16)

        def row_cr(i):
            m = jnp.minimum(base + i, last)
            return m // r, m % r

        def in_copy(i, b):
            ci, ri = row_cr(i)
            return pltpu.make_async_copy(
                x_hbm.at[ci, pl.ds(ri, 1)], rowbuf.at[b], isem)

        def out_copy(i, b):
            ci, ri = row_cr(i)
            return pltpu.make_async_copy(
                outbuf.at[b], out_hbm.at[ci, pl.ds(ri, 1)], osem)

        def gather_row(b):
            def body(g, _):
                o = lane + g * 16
                pos = (num * o + half) // den
                vals = plsc.load_gather(
                    rowbuf.at[b], [jnp.zeros((16,), jnp.int32), pos])
                outbuf[b, 0, pl.ds(g * 16, 16)] = vals
                return 0
            lax.fori_loop(0, NUM_OUT // 16, body, 0, unroll=4)

        in_copy(0, 0).start()
        in_copy(1, 1).start()

        def step(q, _):
            for b in range(2):
                i = q * 2 + b
                in_copy(i, b).wait()

                @pl.when(q > 0)
                def _():
                    out_copy(i - 2, b).wait()

                gather_row(b)
                out_copy(i, b).start()

                @pl.when(q < nrows // 2 - 1)
                def _():
                    in_copy(i + 2, b).start()
            return 0

        lax.fori_loop(0, nrows // 2, step, 0)
        out_copy(nrows - 2, 0).wait()
        out_copy(nrows - 1, 1).wait()

    return k(x_t)


def kernel(landmarks):
    x_t = jnp.transpose(landmarks, (2, 1, 0))
    out_t = _sc_lane_subsample(x_t)
    return jnp.transpose(out_t, (2, 1, 0))


# speedup vs baseline: 193.8394x; 2.9106x over previous
import functools

import jax
import jax.numpy as jnp
from jax import lax
from jax.experimental import pallas as pl
from jax.experimental.pallas import tpu as pltpu
from jax.experimental.pallas import tpu_sc as plsc

NUM_OUT = 1024
NUM_WORKERS = 32
CHUNK = 8
HALF_T = 4096
HALF_O = 512
CH_PER_SLAB = 68
N_TASKS = 3 * CH_PER_SLAB * 2
STEPS = 14


def _sc_lane_subsample(x_t):
    c, r, t = x_t.shape
    num = 2 * (t - 1)
    den = 2 * (NUM_OUT - 1)
    hlf = NUM_OUT - 1
    mesh = plsc.VectorSubcoreMesh(core_axis_name="c", subcore_axis_name="s")

    @functools.partial(
        pl.kernel,
        mesh=mesh,
        out_type=jax.ShapeDtypeStruct((c, r, NUM_OUT), x_t.dtype),
        scratch_types=[
            pltpu.VMEM((2, CHUNK, HALF_T), x_t.dtype),
            pltpu.VMEM((2, CHUNK, HALF_O), x_t.dtype),
            pltpu.SemaphoreType.DMA,
            pltpu.SemaphoreType.DMA,
        ],
        compiler_params=pltpu.CompilerParams(needs_layout_passes=False),
    )
    def k(x_hbm, out_hbm, rowbuf, outbuf, isem, osem):
        wid = lax.axis_index("s") * 2 + lax.axis_index("c")
        lane = lax.iota(jnp.int32, 16)

        def task_decode(i):
            tau = jnp.minimum(wid + NUM_WORKERS * i, N_TASKS - 1)
            slab = tau // (2 * CH_PER_SLAB)
            rem = tau % (2 * CH_PER_SLAB)
            pos = rem // 2
            half = rem % 2
            rstart = pl.multiple_of(pos * CHUNK, CHUNK)
            return slab, rstart, half

        def in_copy(i, b):
            ci, ri, hi = task_decode(i)
            return pltpu.make_async_copy(
                x_hbm.at[ci, pl.ds(ri, CHUNK), pl.ds(hi * HALF_T, HALF_T)],
                rowbuf.at[b], isem)

        def out_copy(i, b):
            ci, ri, hi = task_decode(i)
            return pltpu.make_async_copy(
                outbuf.at[b],
                out_hbm.at[ci, pl.ds(ri, CHUNK), pl.ds(hi * HALF_O, HALF_O)],
                osem)

        def gather_chunk(i, b):
            _, _, hi = task_decode(i)
            obase = hi * HALF_O
            pbase = hi * HALF_T

            def body(h, _):
                j = h // (HALF_O // 16)
                g = h % (HALF_O // 16)
                o = lane + g * 16 + obase
                pos = (num * o + hlf) // den - pbase
                jv = jnp.zeros((16,), jnp.int32) + j
                vals = plsc.load_gather(rowbuf.at[b], [jv, pos])
                outbuf[b, j, pl.ds(g * 16, 16)] = vals
                return 0

            lax.fori_loop(0, CHUNK * (HALF_O // 16), body, 0, unroll=4)

        in_copy(0, 0).start()
        in_copy(1, 1).start()

        def step(q, _):
            for b in range(2):
                i = q * 2 + b
                in_copy(i, b).wait()

                @pl.when(q > 0)
                def _():
                    out_copy(i - 2, b).wait()

                gather_chunk(i, b)
                out_copy(i, b).start()

                @pl.when(q < STEPS // 2 - 1)
                def _():
                    in_copy(i + 2, b).start()
            return 0

        lax.fori_loop(0, STEPS // 2, step, 0)
        out_copy(STEPS - 2, 0).wait()
        out_copy(STEPS - 1, 1).wait()

    return k(x_t)


def kernel(landmarks):
    x_t = jnp.transpose(landmarks, (2, 1, 0))
    out_t = _sc_lane_subsample(x_t)
    return jnp.transpose(out_t, (2, 1, 0))
